# baseline (speedup 1.0000x reference)
import jax
import jax.numpy as jnp
from jax import lax
from jax.experimental import pallas as pl
from jax.experimental.pallas import tpu as pltpu
from jax.experimental.pallas import tpu_sc as plsc

NC = 2
NS = 16
NW = NC * NS
L = 16

B = 2
CH = 128
C = 44
H = W = D = 40
WG = 5
GROUPS = H * (W // WG)
G_PER_W = GROUPS // NW
GROWS = WG * D
OUT_ROWS = B * H * W * D

PAT_OFFS = (0, 16, 32, 48, 64, 72)
DEP_OFFS = (88, 104, 112)


def _body(tbl_hbm, out_hbm, tbl_v, buf0, buf1, buf2, buf3, sem0, sem1, sem2, sem3):
    cid = lax.axis_index("c")
    sid = lax.axis_index("s")
    wid = sid * NC + cid

    pltpu.sync_copy(tbl_hbm, tbl_v)
    lanes = lax.broadcasted_iota(jnp.int32, (L,), 0)

    bufs = (buf0, buf1, buf2, buf3)
    sems = (sem0, sem1, sem2, sem3)

    for buf in bufs:
        def dinit(r, carry):
            d_row = jnp.full((L,), 100 + lax.rem(r, D), jnp.int32)
            for a in DEP_OFFS:
                cols = (a - 88) + lanes
                buf[r, pl.ds(a, L)] = plsc.load_gather(tbl_v, [d_row, cols])
            return carry

        lax.fori_loop(0, GROWS, dinit, 0)

    for i in range(G_PER_W):
        buf = bufs[i % 4]
        sem = sems[i % 4]
        if i >= 4:
            pltpu.make_async_copy(out_hbm.at[pl.ds(0, GROWS)], buf, sem).wait()
            pltpu.make_async_copy(out_hbm.at[pl.ds(0, GROWS)], buf, sem).wait()

        gg = wid * G_PER_W + i
        h = gg // (W // WG)
        w0 = (gg - h * (W // WG)) * WG

        def sub_body(sub, carry):
            w = w0 + sub
            vs = []
            for a in PAT_OFFS:
                p = a + lanes
                in_col = p < C
                ridx = jnp.where(in_col, jnp.full((L,), w, jnp.int32),
                                 jnp.full((L,), 50 + h, jnp.int32))
                cidx = jnp.where(in_col, p, p - C)
                vs.append(plsc.load_gather(tbl_v, [ridx, cidx]))

            def fill(rr, inner):
                r = sub * D + rr
                for a, v in zip(PAT_OFFS, vs):
                    buf[r, pl.ds(a, L)] = v
                return inner

            lax.fori_loop(0, D, fill, 0)
            return carry

        lax.fori_loop(0, WG, sub_body, 0)

        base = (h * W + w0) * D
        pltpu.make_async_copy(buf, out_hbm.at[pl.ds(base, GROWS)], sem).start()
        pltpu.make_async_copy(buf, out_hbm.at[pl.ds(H * W * D + base, GROWS)], sem).start()

    for i in range(G_PER_W - 4, G_PER_W):
        buf = bufs[i % 4]
        sem = sems[i % 4]
        pltpu.make_async_copy(out_hbm.at[pl.ds(0, GROWS)], buf, sem).wait()
        pltpu.make_async_copy(out_hbm.at[pl.ds(0, GROWS)], buf, sem).wait()


@jax.jit
def _pos_embed(tbl):
    mesh = plsc.VectorSubcoreMesh(core_axis_name="c", subcore_axis_name="s")
    f = pl.kernel(
        _body,
        out_type=jax.ShapeDtypeStruct((OUT_ROWS, CH), jnp.float32),
        mesh=mesh,
        compiler_params=pltpu.CompilerParams(needs_layout_passes=False),
        scratch_types=[
            pltpu.VMEM((150, C), jnp.float32),
            pltpu.VMEM((GROWS, CH), jnp.float32),
            pltpu.VMEM((GROWS, CH), jnp.float32),
            pltpu.VMEM((GROWS, CH), jnp.float32),
            pltpu.VMEM((GROWS, CH), jnp.float32),
            pltpu.SemaphoreType.DMA,
            pltpu.SemaphoreType.DMA,
            pltpu.SemaphoreType.DMA,
            pltpu.SemaphoreType.DMA,
        ],
    )
    out = f(tbl)
    return out.reshape(B, H, W, D, CH).transpose(0, 4, 1, 2, 3)


def kernel(x, row_embed, col_embed, depth_embed):
    tbl = jnp.concatenate([col_embed, row_embed, depth_embed], axis=0)
    return _pos_embed(tbl)


# speedup vs baseline: 1.0906x; 1.0906x over previous
import jax
import jax.numpy as jnp
from jax import lax
from jax.experimental import pallas as pl
from jax.experimental.pallas import tpu as pltpu
from jax.experimental.pallas import tpu_sc as plsc

NC = 2
NS = 16
NW = NC * NS
L = 16

B = 2
CH = 128
C = 44
H = W = D = 40
WG = 5
NWG = W // WG
H_PER_W = H * NWG // NW
GROWS = WG * D
OUT_ROWS = B * H * W * D

COL_OFFS = (0, 16, 28)
ROW_OFFS = (44, 60, 72)
DEP_OFFS = (88, 104, 112)
RUN = 4


def _body(tbl_hbm, out_hbm, tbl_v, buf0, buf1, sem0, sem1):
    cid = lax.axis_index("c")
    sid = lax.axis_index("s")
    wid = sid * NC + cid

    pltpu.sync_copy(tbl_hbm, tbl_v)
    lanes = lax.broadcasted_iota(jnp.int32, (L,), 0)

    wg = lax.rem(wid, NWG)
    w0 = wg * WG
    h0 = (wid // NWG) * H_PER_W

    for buf in (buf0, buf1):
        def sub_init(sub, carry):
            w_row = jnp.full((L,), w0 + sub, jnp.int32)
            cvs = [plsc.load_gather(tbl_v, [w_row, a + lanes]) for a in COL_OFFS]

            def crow(rr, inner):
                r = sub * D + rr
                for a, v in zip(COL_OFFS, cvs):
                    buf[r, pl.ds(a, L)] = v
                return inner

            lax.fori_loop(0, D, crow, 0)
            return carry

        lax.fori_loop(0, WG, sub_init, 0)

        def dinit(r, carry):
            d_row = jnp.full((L,), 100 + lax.rem(r, D), jnp.int32)
            for a in DEP_OFFS:
                buf[r, pl.ds(a, L)] = plsc.load_gather(tbl_v, [d_row, (a - 88) + lanes])
            return carry

        lax.fori_loop(0, GROWS, dinit, 0)

    bufs = (buf0, buf1)
    sems = (sem0, sem1)

    for i in range(H_PER_W):
        buf = bufs[i % 2]
        sem = sems[i % 2]
        if i >= 2:
            pltpu.make_async_copy(out_hbm.at[pl.ds(0, GROWS)], buf, sem).wait()
            pltpu.make_async_copy(out_hbm.at[pl.ds(0, GROWS)], buf, sem).wait()

        h = h0 + i
        h_row = jnp.full((L,), 50 + h, jnp.int32)
        rvs = [plsc.load_gather(tbl_v, [h_row, (a - C) + lanes]) for a in ROW_OFFS]

        def fill(g, carry):
            for u in range(RUN):
                r = g * RUN + u
                for a, v in zip(ROW_OFFS, rvs):
                    buf[r, pl.ds(a, L)] = v
            return carry

        lax.fori_loop(0, GROWS // RUN, fill, 0)

        base = (h * W + w0) * D
        pltpu.make_async_copy(buf, out_hbm.at[pl.ds(base, GROWS)], sem).start()
        pltpu.make_async_copy(buf, out_hbm.at[pl.ds(H * W * D + base, GROWS)], sem).start()

    for i in (H_PER_W - 2, H_PER_W - 1):
        buf = bufs[i % 2]
        sem = sems[i % 2]
        pltpu.make_async_copy(out_hbm.at[pl.ds(0, GROWS)], buf, sem).wait()
        pltpu.make_async_copy(out_hbm.at[pl.ds(0, GROWS)], buf, sem).wait()


@jax.jit
def _pos_embed(tbl):
    mesh = plsc.VectorSubcoreMesh(core_axis_name="c", subcore_axis_name="s")
    f = pl.kernel(
        _body,
        out_type=jax.ShapeDtypeStruct((OUT_ROWS, CH), jnp.float32),
        mesh=mesh,
        compiler_params=pltpu.CompilerParams(needs_layout_passes=False),
        scratch_types=[
            pltpu.VMEM((150, C), jnp.float32),
            pltpu.VMEM((GROWS, CH), jnp.float32),
            pltpu.VMEM((GROWS, CH), jnp.float32),
            pltpu.SemaphoreType.DMA,
            pltpu.SemaphoreType.DMA,
        ],
    )
    out = f(tbl)
    return out.reshape(B, H, W, D, CH).transpose(0, 4, 1, 2, 3)


def kernel(x, row_embed, col_embed, depth_embed):
    tbl = jnp.concatenate([col_embed, row_embed, depth_embed], axis=0)
    return _pos_embed(tbl)


# speedup vs baseline: 1.1004x; 1.0089x over previous
import jax
import jax.numpy as jnp
from jax import lax
from jax.experimental import pallas as pl
from jax.experimental.pallas import tpu as pltpu
from jax.experimental.pallas import tpu_sc as plsc

NC = 2
NS = 16
NW = NC * NS
L = 16

B = 2
CH = 128
C = 44
H = W = D = 40
WG = 5
NWG = W // WG
H_PER_W = H * NWG // NW
GROWS = WG * D
OUT_ROWS = B * H * W * D

COL_OFFS = (0, 16, 28)
ROW_OFFS = (44, 60, 72)
DEP_OFFS = (88, 104, 112)
RUN = 4


def _body(col_hbm, row_hbm, dep_hbm, out_hbm, tcol, trow, tdep, buf0, buf1,
          sem0, sem1, sems_):
    cid = lax.axis_index("c")
    sid = lax.axis_index("s")
    wid = sid * NC + cid

    pltpu.make_async_copy(col_hbm, tcol, sems_).start()
    pltpu.make_async_copy(row_hbm, trow, sems_).start()
    pltpu.make_async_copy(dep_hbm, tdep, sems_).start()
    pltpu.make_async_copy(col_hbm, tcol, sems_).wait()
    pltpu.make_async_copy(row_hbm, trow, sems_).wait()
    pltpu.make_async_copy(dep_hbm, tdep, sems_).wait()
    lanes = lax.broadcasted_iota(jnp.int32, (L,), 0)

    wg = lax.rem(wid, NWG)
    w0 = wg * WG
    h0 = (wid // NWG) * H_PER_W

    def static_init(buf):
        def sub_init(sub, carry):
            w_row = jnp.full((L,), w0 + sub, jnp.int32)
            cvs = [plsc.load_gather(tcol, [w_row, a + lanes]) for a in COL_OFFS]

            def crow(rr, inner):
                r = sub * D + rr
                for a, v in zip(COL_OFFS, cvs):
                    buf[r, pl.ds(a, L)] = v
                return inner

            lax.fori_loop(0, D, crow, 0)
            return carry

        lax.fori_loop(0, WG, sub_init, 0)

        def dinit(r, carry):
            d_row = jnp.full((L,), lax.rem(r, D), jnp.int32)
            for a in DEP_OFFS:
                buf[r, pl.ds(a, L)] = plsc.load_gather(tdep, [d_row, (a - 88) + lanes])
            return carry

        lax.fori_loop(0, GROWS, dinit, 0)

    static_init(buf0)

    bufs = (buf0, buf1)
    sems = (sem0, sem1)

    for i in range(H_PER_W):
        buf = bufs[i % 2]
        sem = sems[i % 2]
        if i == 1:
            static_init(buf1)
        if i >= 2:
            pltpu.make_async_copy(out_hbm.at[pl.ds(0, GROWS)], buf, sem).wait()
            pltpu.make_async_copy(out_hbm.at[pl.ds(0, GROWS)], buf, sem).wait()

        h = h0 + i
        h_row = jnp.full((L,), h, jnp.int32)
        rvs = [plsc.load_gather(trow, [h_row, (a - C) + lanes]) for a in ROW_OFFS]

        def fill(g, carry):
            for u in range(RUN):
                r = g * RUN + u
                for a, v in zip(ROW_OFFS, rvs):
                    buf[r, pl.ds(a, L)] = v
            return carry

        lax.fori_loop(0, GROWS // RUN, fill, 0)

        base = (h * W + w0) * D
        pltpu.make_async_copy(buf, out_hbm.at[pl.ds(base, GROWS)], sem).start()
        pltpu.make_async_copy(buf, out_hbm.at[pl.ds(H * W * D + base, GROWS)], sem).start()

    for i in (H_PER_W - 2, H_PER_W - 1):
        buf = bufs[i % 2]
        sem = sems[i % 2]
        pltpu.make_async_copy(out_hbm.at[pl.ds(0, GROWS)], buf, sem).wait()
        pltpu.make_async_copy(out_hbm.at[pl.ds(0, GROWS)], buf, sem).wait()


@jax.jit
def _pos_embed(col_embed, row_embed, depth_embed):
    mesh = plsc.VectorSubcoreMesh(core_axis_name="c", subcore_axis_name="s")
    f = pl.kernel(
        _body,
        out_type=jax.ShapeDtypeStruct((OUT_ROWS, CH), jnp.float32),
        mesh=mesh,
        compiler_params=pltpu.CompilerParams(needs_layout_passes=False),
        scratch_types=[
            pltpu.VMEM((50, C), jnp.float32),
            pltpu.VMEM((50, C), jnp.float32),
            pltpu.VMEM((50, C), jnp.float32),
            pltpu.VMEM((GROWS, CH), jnp.float32),
            pltpu.VMEM((GROWS, CH), jnp.float32),
            pltpu.SemaphoreType.DMA,
            pltpu.SemaphoreType.DMA,
            pltpu.SemaphoreType.DMA,
        ],
    )
    out = f(col_embed, row_embed, depth_embed)
    return out.reshape(B, H, W, D, CH).transpose(0, 4, 1, 2, 3)


def kernel(x, row_embed, col_embed, depth_embed):
    return _pos_embed(col_embed, row_embed, depth_embed)


# speedup vs baseline: 1.1253x; 1.0227x over previous
import jax
import jax.numpy as jnp
from jax import lax
from jax.experimental import pallas as pl
from jax.experimental.pallas import tpu as pltpu
from jax.experimental.pallas import tpu_sc as plsc

NC = 2
NS = 16
NW = NC * NS
L = 16

B = 2
CH = 128
C = 44
H = W = D = 40
WG = 5
NWG = W // WG
H_PER_W = H * NWG // NW
GROWS = WG * D
OUT_ROWS = B * H * W * D

COL_OFFS = (0, 16, 28)
ROW_OFFS = (44, 60, 72)
DEP_OFFS = (88, 104, 112)
RUN = 4


def _body(tbl_hbm, out_hbm, tbl_v, buf0, buf1, sem0, sem1):
    cid = lax.axis_index("c")
    sid = lax.axis_index("s")
    wid = sid * NC + cid

    pltpu.sync_copy(tbl_hbm, tbl_v)
    lanes = lax.broadcasted_iota(jnp.int32, (L,), 0)

    wg = lax.rem(wid, NWG)
    w0 = wg * WG
    h0 = (wid // NWG) * H_PER_W

    def static_init(buf):
        def sub_init(sub, carry):
            w_row = jnp.full((L,), w0 + sub, jnp.int32)
            cvs = [plsc.load_gather(tbl_v, [w_row, a + lanes]) for a in COL_OFFS]

            def crow(rr, inner):
                r = sub * D + rr
                for a, v in zip(COL_OFFS, cvs):
                    buf[r, pl.ds(a, L)] = v
                return inner

            lax.fori_loop(0, D, crow, 0)
            return carry

        lax.fori_loop(0, WG, sub_init, 0)

        def dinit(r, carry):
            d_row = jnp.full((L,), 100 + lax.rem(r, D), jnp.int32)
            for a in DEP_OFFS:
                buf[r, pl.ds(a, L)] = plsc.load_gather(tbl_v, [d_row, (a - 88) + lanes])
            return carry

        lax.fori_loop(0, GROWS, dinit, 0)

    static_init(buf0)

    bufs = (buf0, buf1)
    sems = (sem0, sem1)

    for i in range(H_PER_W):
        buf = bufs[i % 2]
        sem = sems[i % 2]
        if i == 1:
            static_init(buf1)
        if i >= 2:
            pltpu.make_async_copy(out_hbm.at[pl.ds(0, GROWS)], buf, sem).wait()
            pltpu.make_async_copy(out_hbm.at[pl.ds(0, GROWS)], buf, sem).wait()

        h = h0 + i
        h_row = jnp.full((L,), 50 + h, jnp.int32)
        rvs = [plsc.load_gather(tbl_v, [h_row, (a - C) + lanes]) for a in ROW_OFFS]

        def fill(g, carry):
            for u in range(RUN):
                r = g * RUN + u
                for a, v in zip(ROW_OFFS, rvs):
                    buf[r, pl.ds(a, L)] = v
            return carry

        lax.fori_loop(0, GROWS // RUN, fill, 0)

        base = (h * W + w0) * D
        pltpu.make_async_copy(buf, out_hbm.at[pl.ds(base, GROWS)], sem).start()
        pltpu.make_async_copy(buf, out_hbm.at[pl.ds(H * W * D + base, GROWS)], sem).start()

    for i in (H_PER_W - 2, H_PER_W - 1):
        buf = bufs[i % 2]
        sem = sems[i % 2]
        pltpu.make_async_copy(out_hbm.at[pl.ds(0, GROWS)], buf, sem).wait()
        pltpu.make_async_copy(out_hbm.at[pl.ds(0, GROWS)], buf, sem).wait()


@jax.jit
def _pos_embed(tbl):
    mesh = plsc.VectorSubcoreMesh(core_axis_name="c", subcore_axis_name="s")
    f = pl.kernel(
        _body,
        out_type=jax.ShapeDtypeStruct((OUT_ROWS, CH), jnp.float32),
        mesh=mesh,
        compiler_params=pltpu.CompilerParams(needs_layout_passes=False),
        scratch_types=[
            pltpu.VMEM((160, CH), jnp.float32),
            pltpu.VMEM((GROWS, CH), jnp.float32),
            pltpu.VMEM((GROWS, CH), jnp.float32),
            pltpu.SemaphoreType.DMA,
            pltpu.SemaphoreType.DMA,
        ],
    )
    out = f(tbl)
    return out.reshape(B, H, W, D, CH).transpose(0, 4, 1, 2, 3)


def kernel(x, row_embed, col_embed, depth_embed):
    tbl = jnp.pad(jnp.concatenate([col_embed, row_embed, depth_embed], axis=0),
                  ((0, 10), (0, CH - C)))
    return _pos_embed(tbl)
